# Initial kernel scaffold; baseline (speedup 1.0000x reference)
import jax
import jax.numpy as jnp
from jax.experimental import pallas as pl


def kernel(x, padding_mask, wpe, l0_ln1g, l0_ln1b, l0_attnw, l0_attnb, l0_projw, l0_projb, l0_ln2g, l0_ln2b, l0_fcw, l0_fcb, l0_fc2w, l0_fc2b, l1_ln1g, l1_ln1b, l1_attnw, l1_attnb, l1_projw, l1_projb, l1_ln2g, l1_ln2b, l1_fcw, l1_fcb, l1_fc2w, l1_fc2b, l2_ln1g, l2_ln1b, l2_attnw, l2_attnb, l2_projw, l2_projb, l2_ln2g, l2_ln2b, l2_fcw, l2_fcb, l2_fc2w, l2_fc2b, l3_ln1g, l3_ln1b, l3_attnw, l3_attnb, l3_projw, l3_projb, l3_ln2g, l3_ln2b, l3_fcw, l3_fcb, l3_fc2w, l3_fc2b, lnf_g, lnf_b, final_w, final_b):
    raise NotImplementedError("write your pallas kernel here")


# speedup vs baseline: 1.0597x; 1.0597x over previous
import functools
import math

import jax
import jax.numpy as jnp
from jax.experimental import pallas as pl
from jax.experimental.pallas import tpu as pltpu

_MASK_NEG = -1e30
_H = 8
_MB = 2


def _layer_norm(h, g, b, eps=1e-5):
    mu = jnp.mean(h, axis=-1, keepdims=True)
    d = h - mu
    var = jnp.mean(d * d, axis=-1, keepdims=True)
    return d * jax.lax.rsqrt(var + eps) * g + b


def _attention_one(qkv, amask, S, D, H):
    HD = D // H
    scale = 1.0 / math.sqrt(HD)
    ctxs = []
    for h in range(H):
        qh = qkv[:, h * HD:(h + 1) * HD].astype(jnp.bfloat16)
        kh = qkv[:, D + h * HD:D + (h + 1) * HD].astype(jnp.bfloat16)
        vh = qkv[:, 2 * D + h * HD:2 * D + (h + 1) * HD].astype(jnp.bfloat16)
        sh = jax.lax.dot_general(qh, kh, (((1,), (1,)), ((), ())),
                                 preferred_element_type=jnp.float32)
        sh = sh * scale + amask
        sh = sh - jnp.max(sh, axis=-1, keepdims=True)
        ph = jnp.exp(sh)
        ph = ph * pl.reciprocal(jnp.sum(ph, axis=-1, keepdims=True), approx=True)
        ctxs.append(jax.lax.dot_general(ph.astype(jnp.bfloat16), vh,
                                        (((1,), (0,)), ((), ())),
                                        preferred_element_type=jnp.float32))
    return jnp.concatenate(ctxs, axis=-1).astype(jnp.bfloat16)


def _make_block_body(S, D, H, MB, with_final):
    def body(x_ref, mask_ref,
             ln1g_ref, ln1b_ref, attnw_ref, attnb_ref, projw_ref, projb_ref,
             ln2g_ref, ln2b_ref, fcw_ref, fcb_ref, fc2w_ref, fc2b_ref,
             *final_refs):
        o_ref = final_refs[-1]
        x = x_ref[...].reshape(MB * S, D).astype(jnp.float32)

        row = jax.lax.broadcasted_iota(jnp.int32, (S, S), 0)
        col = jax.lax.broadcasted_iota(jnp.int32, (S, S), 1)
        causal = col <= row

        h1 = _layer_norm(x, ln1g_ref[0], ln1b_ref[0]).astype(jnp.bfloat16)
        qkv = jnp.dot(h1, attnw_ref[...],
                      preferred_element_type=jnp.float32) + attnb_ref[0]
        ctxs = []
        for m in range(MB):
            km = mask_ref[m]
            amask = jnp.where(causal & (km > 0.5), 0.0, _MASK_NEG)
            ctxs.append(_attention_one(qkv[m * S:(m + 1) * S], amask, S, D, H))
        ctx = jnp.concatenate(ctxs, axis=0)
        x = x + (jnp.dot(ctx, projw_ref[...],
                         preferred_element_type=jnp.float32) + projb_ref[0])

        h2 = _layer_norm(x, ln2g_ref[0], ln2b_ref[0]).astype(jnp.bfloat16)
        hm = jnp.dot(h2, fcw_ref[...],
                     preferred_element_type=jnp.float32) + fcb_ref[0]
        c = 0.7978845608028654
        hm = 0.5 * hm * (1.0 + jnp.tanh(c * (hm + 0.044715 * hm * hm * hm)))
        mlp = jnp.dot(hm.astype(jnp.bfloat16), fc2w_ref[...],
                      preferred_element_type=jnp.float32) + fc2b_ref[0]
        x = x + mlp

        if with_final:
            lnfg_ref, lnfb_ref, fw_ref, fb_ref = final_refs[:4]
            OUTP = fw_ref.shape[1]
            hf = _layer_norm(x, lnfg_ref[0], lnfb_ref[0])
            hf = jnp.maximum(hf, 0.0).astype(jnp.bfloat16)
            y = jnp.dot(hf, fw_ref[...],
                        preferred_element_type=jnp.float32) + fb_ref[0]
            o_ref[...] = jnp.tanh(y).reshape(MB, S, OUTP)
        else:
            o_ref[...] = x.astype(o_ref.dtype).reshape(MB, S, D)
    return body


def _wspec(shape):
    idx = lambda j: (0,) * len(shape)
    try:
        return pl.BlockSpec(shape, idx, pipeline_mode=pl.Buffered(1))
    except TypeError:
        return pl.BlockSpec(shape, idx)


def _block_call(h, km, layer_params, final_params, H, MB):
    B, S, D = h.shape
    with_final = final_params is not None
    in_specs = [
        pl.BlockSpec((MB, S, D), lambda j: (j, 0, 0)),
        pl.BlockSpec((MB, 1, S), lambda j: (j, 0, 0)),
        _wspec((1, D)), _wspec((1, D)),
        _wspec((D, 3 * D)), _wspec((1, 3 * D)),
        _wspec((D, D)), _wspec((1, D)),
        _wspec((1, D)), _wspec((1, D)),
        _wspec((D, 4 * D)), _wspec((1, 4 * D)),
        _wspec((4 * D, D)), _wspec((1, D)),
    ]
    args = list(layer_params)
    if with_final:
        lnfg, lnfb, fw, fb = final_params
        OUTP = fw.shape[1]
        in_specs += [_wspec((1, D)), _wspec((1, D)),
                     _wspec((D, OUTP)), _wspec((1, OUTP))]
        args += [lnfg, lnfb, fw, fb]
        out_shape = jax.ShapeDtypeStruct((B, S, OUTP), jnp.float32)
        out_spec = pl.BlockSpec((MB, S, OUTP), lambda j: (j, 0, 0))
    else:
        out_shape = jax.ShapeDtypeStruct((B, S, D), jnp.bfloat16)
        out_spec = pl.BlockSpec((MB, S, D), lambda j: (j, 0, 0))

    return pl.pallas_call(
        _make_block_body(S, D, H, MB, with_final),
        out_shape=out_shape,
        grid=(B // MB,),
        in_specs=in_specs,
        out_specs=out_spec,
        compiler_params=pltpu.CompilerParams(
            dimension_semantics=("arbitrary",),
            vmem_limit_bytes=64 * 1024 * 1024,
        ),
    )(h, km, *args)


@jax.jit
def kernel(x, padding_mask, wpe,
           l0_ln1g, l0_ln1b, l0_attnw, l0_attnb, l0_projw, l0_projb,
           l0_ln2g, l0_ln2b, l0_fcw, l0_fcb, l0_fc2w, l0_fc2b,
           l1_ln1g, l1_ln1b, l1_attnw, l1_attnb, l1_projw, l1_projb,
           l1_ln2g, l1_ln2b, l1_fcw, l1_fcb, l1_fc2w, l1_fc2b,
           l2_ln1g, l2_ln1b, l2_attnw, l2_attnb, l2_projw, l2_projb,
           l2_ln2g, l2_ln2b, l2_fcw, l2_fcb, l2_fc2w, l2_fc2b,
           l3_ln1g, l3_ln1b, l3_attnw, l3_attnb, l3_projw, l3_projb,
           l3_ln2g, l3_ln2b, l3_fcw, l3_fcb, l3_fc2w, l3_fc2b,
           lnf_g, lnf_b, final_w, final_b):
    B, S, D = x.shape
    h = (x + wpe[None, :S, :]).astype(jnp.bfloat16)
    km = padding_mask.astype(jnp.float32)[:, None, :]

    layers = [
        (l0_ln1g, l0_ln1b, l0_attnw, l0_attnb, l0_projw, l0_projb,
         l0_ln2g, l0_ln2b, l0_fcw, l0_fcb, l0_fc2w, l0_fc2b),
        (l1_ln1g, l1_ln1b, l1_attnw, l1_attnb, l1_projw, l1_projb,
         l1_ln2g, l1_ln2b, l1_fcw, l1_fcb, l1_fc2w, l1_fc2b),
        (l2_ln1g, l2_ln1b, l2_attnw, l2_attnb, l2_projw, l2_projb,
         l2_ln2g, l2_ln2b, l2_fcw, l2_fcb, l2_fc2w, l2_fc2b),
        (l3_ln1g, l3_ln1b, l3_attnw, l3_attnb, l3_projw, l3_projb,
         l3_ln2g, l3_ln2b, l3_fcw, l3_fcb, l3_fc2w, l3_fc2b),
    ]

    OUT = final_w.shape[1]
    OUTP = ((OUT + 127) // 128) * 128
    fw = jnp.pad(final_w, ((0, 0), (0, OUTP - OUT)))
    fb = jnp.pad(final_b, ((0, 0), (0, OUTP - OUT)))

    for lp in layers[:-1]:
        h = _block_call(h, km, lp, None, _H, _MB)
    y = _block_call(h, km, layers[-1], (lnf_g, lnf_b, fw, fb), _H, _MB)
    return y[:, :, :OUT]


# speedup vs baseline: 1.0722x; 1.0118x over previous
import functools
import math

import jax
import jax.numpy as jnp
from jax.experimental import pallas as pl
from jax.experimental.pallas import tpu as pltpu

_MASK_NEG = -1e30
_H = 8
_MB = 2
_L = 4


def _layer_norm(h, g, b, eps=1e-5):
    mu = jnp.mean(h, axis=-1, keepdims=True)
    d = h - mu
    var = jnp.mean(d * d, axis=-1, keepdims=True)
    return d * jax.lax.rsqrt(var + eps) * g + b


def _attention_one(qkv, amask, S, D, H):
    HD = D // H
    scale = 1.0 / math.sqrt(HD)
    ctxs = []
    for h in range(H):
        qh = qkv[:, h * HD:(h + 1) * HD].astype(jnp.bfloat16)
        kh = qkv[:, D + h * HD:D + (h + 1) * HD].astype(jnp.bfloat16)
        vh = qkv[:, 2 * D + h * HD:2 * D + (h + 1) * HD].astype(jnp.bfloat16)
        sh = jax.lax.dot_general(qh, kh, (((1,), (1,)), ((), ())),
                                 preferred_element_type=jnp.float32)
        sh = sh * scale + amask
        sh = sh - jnp.max(sh, axis=-1, keepdims=True)
        ph = jnp.exp(sh)
        ph = ph * pl.reciprocal(jnp.sum(ph, axis=-1, keepdims=True), approx=True)
        ctxs.append(jax.lax.dot_general(ph.astype(jnp.bfloat16), vh,
                                        (((1,), (0,)), ((), ())),
                                        preferred_element_type=jnp.float32))
    return jnp.concatenate(ctxs, axis=-1).astype(jnp.bfloat16)


def _make_body(S, D, H, MB, L):
    def body(h_ref, mask_ref,
             ln1g_ref, ln1b_ref, attnb_ref, projb_ref,
             ln2g_ref, ln2b_ref, fcb_ref, fc2b_ref,
             lnfg_ref, lnfb_ref, fw_ref, fb_ref,
             aw0, aw1, aw2, aw3, pw0, pw1, pw2, pw3,
             fw0, fw1, fw2, fw3, f2w0, f2w1, f2w2, f2w3,
             ho_ref, y_ref,
             wa, wp, wf, wf2, sem_a, sem_p, sem_f, sem_f2):
        l = pl.program_id(0)
        j = pl.program_id(1)
        aws = (aw0, aw1, aw2, aw3)
        pws = (pw0, pw1, pw2, pw3)
        fws = (fw0, fw1, fw2, fw3)
        f2ws = (f2w0, f2w1, f2w2, f2w3)

        @pl.when(j == 0)
        def _():
            for i in range(L):
                @pl.when(l == i)
                def _():
                    pltpu.make_async_copy(aws[i], wa, sem_a).start()
                    pltpu.make_async_copy(pws[i], wp, sem_p).start()

            @pl.when(l == 0)
            def _():
                pltpu.make_async_copy(fws[0], wf.at[0], sem_f.at[0]).start()
                pltpu.make_async_copy(f2ws[0], wf2.at[0], sem_f2.at[0]).start()

            for i in range(1, L):
                @pl.when(l + 1 == i)
                def _():
                    pltpu.make_async_copy(fws[i], wf.at[i % 2],
                                          sem_f.at[i % 2]).start()
                    pltpu.make_async_copy(f2ws[i], wf2.at[i % 2],
                                          sem_f2.at[i % 2]).start()

            pltpu.make_async_copy(wa, wa, sem_a).wait()
            pltpu.make_async_copy(wp, wp, sem_p).wait()
            buf = l % 2
            pltpu.make_async_copy(wf.at[buf], wf.at[buf], sem_f.at[buf]).wait()
            pltpu.make_async_copy(wf2.at[buf], wf2.at[buf],
                                  sem_f2.at[buf]).wait()

        buf = l % 2

        x = h_ref[...].reshape(MB * S, D).astype(jnp.float32)

        row = jax.lax.broadcasted_iota(jnp.int32, (S, S), 0)
        col = jax.lax.broadcasted_iota(jnp.int32, (S, S), 1)
        causal = col <= row

        h1 = _layer_norm(x, ln1g_ref[0], ln1b_ref[0]).astype(jnp.bfloat16)
        qkv = jnp.dot(h1, wa[...],
                      preferred_element_type=jnp.float32) + attnb_ref[0]
        ctxs = []
        for m in range(MB):
            km = mask_ref[m]
            amask = jnp.where(causal & (km > 0.5), 0.0, _MASK_NEG)
            ctxs.append(_attention_one(qkv[m * S:(m + 1) * S], amask, S, D, H))
        ctx = jnp.concatenate(ctxs, axis=0)
        x = x + (jnp.dot(ctx, wp[...],
                         preferred_element_type=jnp.float32) + projb_ref[0])

        h2 = _layer_norm(x, ln2g_ref[0], ln2b_ref[0]).astype(jnp.bfloat16)
        hm = jnp.dot(h2, wf[buf],
                     preferred_element_type=jnp.float32) + fcb_ref[0]
        c = 0.7978845608028654
        hm = 0.5 * hm * (1.0 + jnp.tanh(c * (hm + 0.044715 * hm * hm * hm)))
        mlp = jnp.dot(hm.astype(jnp.bfloat16), wf2[buf],
                      preferred_element_type=jnp.float32) + fc2b_ref[0]
        x = x + mlp

        ho_ref[...] = x.astype(ho_ref.dtype).reshape(MB, S, D)

        @pl.when(l == L - 1)
        def _():
            OUTP = fw_ref.shape[1]
            hf = _layer_norm(x, lnfg_ref[0], lnfb_ref[0])
            hf = jnp.maximum(hf, 0.0).astype(jnp.bfloat16)
            y = jnp.dot(hf, fw_ref[...],
                        preferred_element_type=jnp.float32) + fb_ref[0]
            y_ref[...] = jnp.tanh(y).reshape(MB, S, OUTP)
    return body


def _small_spec(shape):
    return pl.BlockSpec((1,) + shape, lambda l, j: (l,) + (0,) * len(shape))


def _const_spec(shape):
    idx = lambda l, j: (0,) * len(shape)
    try:
        return pl.BlockSpec(shape, idx, pipeline_mode=pl.Buffered(1))
    except TypeError:
        return pl.BlockSpec(shape, idx)


@jax.jit
def kernel(x, padding_mask, wpe,
           l0_ln1g, l0_ln1b, l0_attnw, l0_attnb, l0_projw, l0_projb,
           l0_ln2g, l0_ln2b, l0_fcw, l0_fcb, l0_fc2w, l0_fc2b,
           l1_ln1g, l1_ln1b, l1_attnw, l1_attnb, l1_projw, l1_projb,
           l1_ln2g, l1_ln2b, l1_fcw, l1_fcb, l1_fc2w, l1_fc2b,
           l2_ln1g, l2_ln1b, l2_attnw, l2_attnb, l2_projw, l2_projb,
           l2_ln2g, l2_ln2b, l2_fcw, l2_fcb, l2_fc2w, l2_fc2b,
           l3_ln1g, l3_ln1b, l3_attnw, l3_attnb, l3_projw, l3_projb,
           l3_ln2g, l3_ln2b, l3_fcw, l3_fcb, l3_fc2w, l3_fc2b,
           lnf_g, lnf_b, final_w, final_b):
    B, S, D = x.shape
    H, MB, L = _H, _MB, _L
    NJ = B // MB

    h0 = (x + wpe[None, :S, :]).astype(jnp.bfloat16)
    km = padding_mask.astype(jnp.float32)[:, None, :]

    ln1g = jnp.stack([l0_ln1g, l1_ln1g, l2_ln1g, l3_ln1g])
    ln1b = jnp.stack([l0_ln1b, l1_ln1b, l2_ln1b, l3_ln1b])
    attnb = jnp.stack([l0_attnb, l1_attnb, l2_attnb, l3_attnb])
    projb = jnp.stack([l0_projb, l1_projb, l2_projb, l3_projb])
    ln2g = jnp.stack([l0_ln2g, l1_ln2g, l2_ln2g, l3_ln2g])
    ln2b = jnp.stack([l0_ln2b, l1_ln2b, l2_ln2b, l3_ln2b])
    fcb = jnp.stack([l0_fcb, l1_fcb, l2_fcb, l3_fcb])
    fc2b = jnp.stack([l0_fc2b, l1_fc2b, l2_fc2b, l3_fc2b])

    OUT = final_w.shape[1]
    OUTP = ((OUT + 127) // 128) * 128
    fw = jnp.pad(final_w, ((0, 0), (0, OUTP - OUT)))
    fb = jnp.pad(final_b, ((0, 0), (0, OUTP - OUT)))

    any_spec = pl.BlockSpec(memory_space=pl.ANY)
    in_specs = [
        pl.BlockSpec((MB, S, D), lambda l, j: (j, 0, 0)),
        pl.BlockSpec((MB, 1, S), lambda l, j: (j, 0, 0)),
        _small_spec((1, D)), _small_spec((1, D)),
        _small_spec((1, 3 * D)), _small_spec((1, D)),
        _small_spec((1, D)), _small_spec((1, D)),
        _small_spec((1, 4 * D)), _small_spec((1, D)),
        _const_spec((1, D)), _const_spec((1, D)),
        _const_spec((D, OUTP)), _const_spec((1, OUTP)),
    ] + [any_spec] * 16

    out_shapes = [
        jax.ShapeDtypeStruct((B, S, D), jnp.bfloat16),
        jax.ShapeDtypeStruct((B, S, OUTP), jnp.float32),
    ]
    out_specs = [
        pl.BlockSpec((MB, S, D), lambda l, j: (j, 0, 0)),
        pl.BlockSpec((MB, S, OUTP),
                     lambda l, j: (jnp.where(l == _L - 1, j, 0), 0, 0)),
    ]

    scratch = [
        pltpu.VMEM((D, 3 * D), jnp.bfloat16),
        pltpu.VMEM((D, D), jnp.bfloat16),
        pltpu.VMEM((2, D, 4 * D), jnp.bfloat16),
        pltpu.VMEM((2, 4 * D, D), jnp.bfloat16),
        pltpu.SemaphoreType.DMA,
        pltpu.SemaphoreType.DMA,
        pltpu.SemaphoreType.DMA((2,)),
        pltpu.SemaphoreType.DMA((2,)),
    ]

    _, y = pl.pallas_call(
        _make_body(S, D, H, MB, L),
        out_shape=out_shapes,
        grid=(L, NJ),
        in_specs=in_specs,
        out_specs=out_specs,
        input_output_aliases={0: 0},
        scratch_shapes=scratch,
        compiler_params=pltpu.CompilerParams(
            dimension_semantics=("arbitrary", "arbitrary"),
            vmem_limit_bytes=64 * 1024 * 1024,
        ),
    )(h0, km, ln1g, ln1b, attnb, projb, ln2g, ln2b, fcb, fc2b,
      lnf_g, lnf_b, fw, fb,
      l0_attnw, l1_attnw, l2_attnw, l3_attnw,
      l0_projw, l1_projw, l2_projw, l3_projw,
      l0_fcw, l1_fcw, l2_fcw, l3_fcw,
      l0_fc2w, l1_fc2w, l2_fc2w, l3_fc2w)
    return y[:, :, :OUT]
